# baseline (speedup 1.0000x reference)
import functools

import jax
import jax.numpy as jnp
from jax import lax
from jax.experimental import pallas as pl
from jax.experimental.pallas import tpu as pltpu
from jax.experimental.pallas import tpu_sc as plsc

_N_L1 = 512
_N_PSQT = 8
_TOTAL = _N_L1 + _N_PSQT
_BATCH = 4096
_ACTIVE = 20
_NC = 2
_NS = 16
_NW = _NC * _NS
_BPW = _BATCH // _NW
_LANES = 16
_NCHUNK = 33
_MAGIC = 12582912.0
_MAGIC20 = _MAGIC * _ACTIVE


def _sc_bag(idx_flat, weight, rinv33, sfin33, bias33):
    mesh = plsc.VectorSubcoreMesh(core_axis_name="c", subcore_axis_name="s")

    @functools.partial(
        pl.kernel,
        out_type=jax.ShapeDtypeStruct((_BATCH, _TOTAL), jnp.float32),
        mesh=mesh,
        compiler_params=pltpu.CompilerParams(use_tc_tiling_on_sc=False),
        scratch_types=[
            pltpu.VMEM((_BPW, _ACTIVE), jnp.int32),
            pltpu.VMEM((_ACTIVE, _TOTAL), jnp.float32),
            pltpu.VMEM((1, _TOTAL), jnp.float32),
            pltpu.VMEM((_NCHUNK, _LANES), jnp.float32),
            pltpu.VMEM((_NCHUNK, _LANES), jnp.float32),
            pltpu.VMEM((_NCHUNK, _LANES), jnp.float32),
        ],
    )
    def body(idx_hbm, w_hbm, rinv_hbm, sfin_hbm, bias_hbm, out_hbm,
             idx_v, rows_v, out_v, rinv_v, sfin_v, bias_v):
        wid = lax.axis_index("s") * _NC + lax.axis_index("c")
        pltpu.sync_copy(idx_hbm.at[pl.ds(wid * _BPW, _BPW)], idx_v)
        pltpu.sync_copy(rinv_hbm, rinv_v)
        pltpu.sync_copy(sfin_hbm, sfin_v)
        pltpu.sync_copy(bias_hbm, bias_v)

        @pl.loop(0, _BPW)
        def _(s):
            pltpu.sync_copy(w_hbm.at[idx_v.at[s]], rows_v)

            rv = rinv_v[32, :]
            acc0 = rows_v[0, pl.ds(504, _LANES)] * rv
            acc1 = rows_v[1, pl.ds(504, _LANES)] * rv
            for a in range(2, _ACTIVE, 2):
                acc0 = acc0 + rows_v[a, pl.ds(504, _LANES)] * rv
                acc1 = acc1 + rows_v[a + 1, pl.ds(504, _LANES)] * rv
            out_v[0, pl.ds(504, _LANES)] = (
                (acc0 + acc1) * sfin_v[32, :] + bias_v[32, :])

            mvec = jnp.full((_LANES,), _MAGIC, jnp.float32)
            m20 = jnp.full((_LANES,), _MAGIC20, jnp.float32)
            for c in range(32):
                c0 = c * _LANES
                rv = rinv_v[c, :]
                acc0 = rows_v[0, pl.ds(c0, _LANES)] * rv + mvec
                acc1 = rows_v[1, pl.ds(c0, _LANES)] * rv + mvec
                for a in range(2, _ACTIVE, 2):
                    acc0 = acc0 + (rows_v[a, pl.ds(c0, _LANES)] * rv + mvec)
                    acc1 = acc1 + (rows_v[a + 1, pl.ds(c0, _LANES)] * rv
                                   + mvec)
                total = (acc0 + acc1) - m20
                out_v[0, pl.ds(c0, _LANES)] = (
                    total * sfin_v[c, :] + bias_v[c, :])

            pltpu.sync_copy(out_v, out_hbm.at[pl.ds(wid * _BPW + s, 1)])

    return body(idx_flat, weight, rinv33, sfin33, bias33)


def kernel(indices, weight, bias, scale_l1, scale_psqt):
    s_full = jnp.concatenate([scale_l1, scale_psqt]).astype(jnp.float32)
    rinv_full = (1.0 / s_full).astype(jnp.float32)
    starts = jnp.array([16 * c for c in range(32)] + [504], jnp.int32)
    cols = starts[:, None] + jnp.arange(16, dtype=jnp.int32)[None, :]
    rinv33 = jnp.take(rinv_full, cols, axis=0)
    sfin33 = jnp.take(s_full, cols, axis=0)
    bias33 = jnp.take(bias.astype(jnp.float32), cols, axis=0)
    return _sc_bag(indices, weight, rinv33, sfin33, bias33)


# speedup vs baseline: 1.0350x; 1.0350x over previous
import functools

import jax
import jax.numpy as jnp
from jax import lax
from jax.experimental import pallas as pl
from jax.experimental.pallas import tpu as pltpu
from jax.experimental.pallas import tpu_sc as plsc

_N_L1 = 512
_N_PSQT = 8
_TOTAL = _N_L1 + _N_PSQT
_BATCH = 4096
_ACTIVE = 20
_NC = 2
_NS = 16
_NW = _NC * _NS
_BPW = _BATCH // _NW
_LANES = 16
_G = 2
_NGRP = _BPW // _G
_GROWS = _G * _ACTIVE
_MAGIC = 12582912.0
_MAGIC20 = _MAGIC * _ACTIVE


def _sc_bag(idx_flat, weight, rinv33, sfin33, bias33):
    mesh = plsc.VectorSubcoreMesh(core_axis_name="c", subcore_axis_name="s")

    @functools.partial(
        pl.kernel,
        out_type=jax.ShapeDtypeStruct((_BATCH, _TOTAL), jnp.float32),
        mesh=mesh,
        compiler_params=pltpu.CompilerParams(use_tc_tiling_on_sc=False),
        scratch_types=[
            pltpu.VMEM((_BPW * _ACTIVE,), jnp.int32),
            pltpu.VMEM((_GROWS, _TOTAL), jnp.float32),
            pltpu.VMEM((_GROWS, _TOTAL), jnp.float32),
            pltpu.VMEM((_G, _TOTAL), jnp.float32),
            pltpu.VMEM((33, _LANES), jnp.float32),
            pltpu.VMEM((33, _LANES), jnp.float32),
            pltpu.VMEM((33, _LANES), jnp.float32),
            pltpu.SemaphoreType.DMA,
            pltpu.SemaphoreType.DMA,
        ],
    )
    def body(idx_hbm, w_hbm, rinv_hbm, sfin_hbm, bias_hbm, out_hbm,
             idx_v, rows0, rows1, out_v, rinv_v, sfin_v, bias_v,
             sem0, sem1):
        wid = lax.axis_index("s") * _NC + lax.axis_index("c")
        nidx = _BPW * _ACTIVE
        pltpu.sync_copy(idx_hbm.at[pl.ds(wid * nidx, nidx)], idx_v)
        pltpu.sync_copy(rinv_hbm, rinv_v)
        pltpu.sync_copy(sfin_hbm, sfin_v)
        pltpu.sync_copy(bias_hbm, bias_v)

        def start_gather(g, rows_v, sem):
            pltpu.async_copy(
                w_hbm.at[idx_v.at[pl.ds(g * _GROWS, _GROWS)]], rows_v, sem)

        def wait_gather(rows_v, sem):
            pltpu.make_async_copy(
                w_hbm.at[idx_v.at[pl.ds(0, _GROWS)]], rows_v, sem).wait()

        def compute_group(g, rows_v):
            for p in range(_G):
                r0 = p * _ACTIVE

                rv = rinv_v[32, :]
                acc0 = rows_v[r0 + 0, pl.ds(504, _LANES)] * rv
                acc1 = rows_v[r0 + 1, pl.ds(504, _LANES)] * rv
                for a in range(2, _ACTIVE, 2):
                    acc0 = acc0 + rows_v[r0 + a, pl.ds(504, _LANES)] * rv
                    acc1 = acc1 + rows_v[r0 + a + 1, pl.ds(504, _LANES)] * rv
                out_v[p, pl.ds(504, _LANES)] = (
                    (acc0 + acc1) * sfin_v[32, :] + bias_v[32, :])

                mvec = jnp.full((_LANES,), _MAGIC, jnp.float32)
                m20 = jnp.full((_LANES,), _MAGIC20, jnp.float32)
                for c in range(32):
                    c0 = c * _LANES
                    rv = rinv_v[c, :]
                    acc0 = rows_v[r0 + 0, pl.ds(c0, _LANES)] * rv + mvec
                    acc1 = rows_v[r0 + 1, pl.ds(c0, _LANES)] * rv + mvec
                    for a in range(2, _ACTIVE, 2):
                        acc0 = acc0 + (rows_v[r0 + a, pl.ds(c0, _LANES)] * rv
                                       + mvec)
                        acc1 = acc1 + (rows_v[r0 + a + 1, pl.ds(c0, _LANES)]
                                       * rv + mvec)
                    total = (acc0 + acc1) - m20
                    out_v[p, pl.ds(c0, _LANES)] = (
                        total * sfin_v[c, :] + bias_v[c, :])

            pltpu.sync_copy(out_v, out_hbm.at[pl.ds(wid * _BPW + g * _G, _G)])

        start_gather(0, rows0, sem0)

        @pl.loop(0, _NGRP, step=2)
        def _(it):
            wait_gather(rows0, sem0)
            start_gather(it + 1, rows1, sem1)
            compute_group(it, rows0)

            wait_gather(rows1, sem1)

            @pl.when(it + 2 < _NGRP)
            def _():
                start_gather(it + 2, rows0, sem0)

            compute_group(it + 1, rows1)

    return body(idx_flat, weight, rinv33, sfin33, bias33)


def kernel(indices, weight, bias, scale_l1, scale_psqt):
    s_full = jnp.concatenate([scale_l1, scale_psqt]).astype(jnp.float32)
    rinv_full = (1.0 / s_full).astype(jnp.float32)
    starts = jnp.array([16 * c for c in range(32)] + [504], jnp.int32)
    cols = starts[:, None] + jnp.arange(16, dtype=jnp.int32)[None, :]
    rinv33 = jnp.take(rinv_full, cols, axis=0)
    sfin33 = jnp.take(s_full, cols, axis=0)
    bias33 = jnp.take(bias.astype(jnp.float32), cols, axis=0)
    idx_flat = indices.reshape(-1)
    return _sc_bag(idx_flat, weight, rinv33, sfin33, bias33)


# speedup vs baseline: 2.7718x; 2.6782x over previous
import functools

import jax
import jax.numpy as jnp
from jax import lax
from jax.experimental import pallas as pl
from jax.experimental.pallas import tpu as pltpu
from jax.experimental.pallas import tpu_sc as plsc

_N_L1 = 512
_N_PSQT = 8
_TOTAL = _N_L1 + _N_PSQT
_BATCH = 4096
_ACTIVE = 20
_NC = 2
_NS = 16
_NW = _NC * _NS
_BPW = _BATCH // _NW
_LANES = 16
_G = 2
_NGRP = _BPW // _G
_GROWS = _G * _ACTIVE
_PSQT_W = 128
_MAGIC = 12582912.0
_MAGIC20 = _MAGIC * _ACTIVE


def _sc_bag(idx_flat, weight, wp128, rinv33, sfin33, bias33):
    mesh = plsc.VectorSubcoreMesh(core_axis_name="c", subcore_axis_name="s")

    @functools.partial(
        pl.kernel,
        out_type=jax.ShapeDtypeStruct((_BATCH, _TOTAL), jnp.float32),
        mesh=mesh,
        scratch_types=[
            pltpu.VMEM((_BPW * _ACTIVE,), jnp.int32),
            pltpu.VMEM((_GROWS, _N_L1), jnp.float32),
            pltpu.VMEM((_GROWS, _N_L1), jnp.float32),
            pltpu.VMEM((_GROWS, _PSQT_W), jnp.float32),
            pltpu.VMEM((_GROWS, _PSQT_W), jnp.float32),
            pltpu.VMEM((_G, _TOTAL), jnp.float32),
            pltpu.VMEM((33, _LANES), jnp.float32),
            pltpu.VMEM((33, _LANES), jnp.float32),
            pltpu.VMEM((33, _LANES), jnp.float32),
            pltpu.SemaphoreType.DMA,
            pltpu.SemaphoreType.DMA,
            pltpu.SemaphoreType.DMA,
            pltpu.SemaphoreType.DMA,
        ],
    )
    def body(idx_hbm, w_hbm, wp_hbm, rinv_hbm, sfin_hbm, bias_hbm, out_hbm,
             idx_v, rows0, rows1, prow0, prow1, out_v, rinv_v, sfin_v, bias_v,
             sem0, sem1, psem0, psem1):
        wid = lax.axis_index("s") * _NC + lax.axis_index("c")
        nidx = _BPW * _ACTIVE
        pltpu.sync_copy(idx_hbm.at[pl.ds(wid * nidx, nidx)], idx_v)
        pltpu.sync_copy(rinv_hbm, rinv_v)
        pltpu.sync_copy(sfin_hbm, sfin_v)
        pltpu.sync_copy(bias_hbm, bias_v)

        def start_gather(g, rows_v, prow_v, sem, psem):
            islc = idx_v.at[pl.ds(g * _GROWS, _GROWS)]
            pltpu.async_copy(w_hbm.at[islc, pl.ds(0, _N_L1)], rows_v, sem)
            pltpu.async_copy(wp_hbm.at[islc], prow_v, psem)

        def wait_gather(rows_v, prow_v, sem, psem):
            islc = idx_v.at[pl.ds(0, _GROWS)]
            pltpu.make_async_copy(
                w_hbm.at[islc, pl.ds(0, _N_L1)], rows_v, sem).wait()
            pltpu.make_async_copy(wp_hbm.at[islc], prow_v, psem).wait()

        def compute_group(g, rows_v, prow_v):
            for p in range(_G):
                r0 = p * _ACTIVE

                rv = rinv_v[32, :]
                acc0 = prow_v[r0 + 0, pl.ds(0, _LANES)] * rv
                acc1 = prow_v[r0 + 1, pl.ds(0, _LANES)] * rv
                for a in range(2, _ACTIVE, 2):
                    acc0 = acc0 + prow_v[r0 + a, pl.ds(0, _LANES)] * rv
                    acc1 = acc1 + prow_v[r0 + a + 1, pl.ds(0, _LANES)] * rv
                out_v[p, pl.ds(504, _LANES)] = (
                    (acc0 + acc1) * sfin_v[32, :] + bias_v[32, :])

                mvec = jnp.full((_LANES,), _MAGIC, jnp.float32)
                m20 = jnp.full((_LANES,), _MAGIC20, jnp.float32)
                for c in range(32):
                    c0 = c * _LANES
                    rv = rinv_v[c, :]
                    acc0 = rows_v[r0 + 0, pl.ds(c0, _LANES)] * rv + mvec
                    acc1 = rows_v[r0 + 1, pl.ds(c0, _LANES)] * rv + mvec
                    for a in range(2, _ACTIVE, 2):
                        acc0 = acc0 + (rows_v[r0 + a, pl.ds(c0, _LANES)] * rv
                                       + mvec)
                        acc1 = acc1 + (rows_v[r0 + a + 1, pl.ds(c0, _LANES)]
                                       * rv + mvec)
                    total = (acc0 + acc1) - m20
                    out_v[p, pl.ds(c0, _LANES)] = (
                        total * sfin_v[c, :] + bias_v[c, :])

            pltpu.sync_copy(out_v, out_hbm.at[pl.ds(wid * _BPW + g * _G, _G)])

        start_gather(0, rows0, prow0, sem0, psem0)

        @pl.loop(0, _NGRP, step=2)
        def _(it):
            wait_gather(rows0, prow0, sem0, psem0)
            start_gather(it + 1, rows1, prow1, sem1, psem1)
            compute_group(it, rows0, prow0)

            wait_gather(rows1, prow1, sem1, psem1)

            @pl.when(it + 2 < _NGRP)
            def _():
                start_gather(it + 2, rows0, prow0, sem0, psem0)

            compute_group(it + 1, rows1, prow1)

    return body(idx_flat, weight, wp128, rinv33, sfin33, bias33)


def kernel(indices, weight, bias, scale_l1, scale_psqt):
    s_full = jnp.concatenate([scale_l1, scale_psqt]).astype(jnp.float32)
    rinv_full = (1.0 / s_full).astype(jnp.float32)
    starts = jnp.array([16 * c for c in range(32)] + [504], jnp.int32)
    cols = starts[:, None] + jnp.arange(16, dtype=jnp.int32)[None, :]
    rinv33 = jnp.take(rinv_full, cols, axis=0)
    sfin33 = jnp.take(s_full, cols, axis=0)
    bias33 = jnp.take(bias.astype(jnp.float32), cols, axis=0)
    idx_flat = indices.reshape(-1)
    wp128 = jnp.pad(weight[:, _N_L1:], ((0, 0), (8, _PSQT_W - 8 - _N_PSQT)))
    return _sc_bag(idx_flat, weight, wp128, rinv33, sfin33, bias33)


# speedup vs baseline: 2.7815x; 1.0035x over previous
import functools

import jax
import jax.numpy as jnp
from jax import lax
from jax.experimental import pallas as pl
from jax.experimental.pallas import tpu as pltpu
from jax.experimental.pallas import tpu_sc as plsc

_N_L1 = 512
_N_PSQT = 8
_TOTAL = _N_L1 + _N_PSQT
_BATCH = 4096
_ACTIVE = 20
_NC = 2
_NS = 16
_NW = _NC * _NS
_BPW = _BATCH // _NW
_LANES = 16
_G = 2
_NGRP = _BPW // _G
_GROWS = _G * _ACTIVE
_PSQT_W = 128
_MAGIC = 12582912.0
_MAGIC20 = _MAGIC * _ACTIVE


def _sc_bag(idx_flat, weight, wp128, rinv33, sfin33, bias33):
    mesh = plsc.VectorSubcoreMesh(core_axis_name="c", subcore_axis_name="s")

    @functools.partial(
        pl.kernel,
        out_type=jax.ShapeDtypeStruct((_BATCH, _TOTAL), jnp.float32),
        mesh=mesh,
        scratch_types=[
            pltpu.VMEM((_BPW * _ACTIVE,), jnp.int32),
            pltpu.VMEM((_GROWS, _N_L1), jnp.float32),
            pltpu.VMEM((_GROWS, _N_L1), jnp.float32),
            pltpu.VMEM((_GROWS, _PSQT_W), jnp.float32),
            pltpu.VMEM((_GROWS, _PSQT_W), jnp.float32),
            pltpu.VMEM((_G, _TOTAL), jnp.float32),
            pltpu.VMEM((33, _LANES), jnp.float32),
            pltpu.VMEM((33, _LANES), jnp.float32),
            pltpu.VMEM((33, _LANES), jnp.float32),
            pltpu.SemaphoreType.DMA,
            pltpu.SemaphoreType.DMA,
            pltpu.SemaphoreType.DMA,
            pltpu.SemaphoreType.DMA,
        ],
    )
    def body(idx_hbm, w_hbm, wp_hbm, rinv_hbm, sfin_hbm, bias_hbm, out_hbm,
             idx_v, rows0, rows1, prow0, prow1, out_v, rinv_v, sfin_v, bias_v,
             sem0, sem1, psem0, psem1):
        wid = lax.axis_index("s") * _NC + lax.axis_index("c")
        nidx = _BPW * _ACTIVE
        pltpu.sync_copy(idx_hbm.at[pl.ds(wid * nidx, nidx)], idx_v)
        pltpu.sync_copy(rinv_hbm, rinv_v)
        pltpu.sync_copy(sfin_hbm, sfin_v)
        pltpu.sync_copy(bias_hbm, bias_v)

        def start_gather(g, rows_v, prow_v, sem, psem):
            islc = idx_v.at[pl.ds(g * _GROWS, _GROWS)]
            pltpu.async_copy(w_hbm.at[islc, pl.ds(0, _N_L1)], rows_v, sem)
            pltpu.async_copy(wp_hbm.at[islc], prow_v, psem)

        def wait_gather(rows_v, prow_v, sem, psem):
            islc = idx_v.at[pl.ds(0, _GROWS)]
            pltpu.make_async_copy(
                w_hbm.at[islc, pl.ds(0, _N_L1)], rows_v, sem).wait()
            pltpu.make_async_copy(wp_hbm.at[islc], prow_v, psem).wait()

        def compute_group(g, rows_v, prow_v):
            for p in range(_G):
                r0 = p * _ACTIVE

                rv = rinv_v[32, :]
                acc0 = prow_v[r0 + 0, pl.ds(0, _LANES)] * rv
                acc1 = prow_v[r0 + 1, pl.ds(0, _LANES)] * rv
                for a in range(2, _ACTIVE, 2):
                    acc0 = acc0 + prow_v[r0 + a, pl.ds(0, _LANES)] * rv
                    acc1 = acc1 + prow_v[r0 + a + 1, pl.ds(0, _LANES)] * rv
                out_v[p, pl.ds(504, _LANES)] = (
                    (acc0 + acc1) * sfin_v[32, :] + bias_v[32, :])

                mvec = jnp.full((_LANES,), _MAGIC, jnp.float32)
                m20 = jnp.full((_LANES,), _MAGIC20, jnp.float32)
                for c in range(32):
                    c0 = c * _LANES
                    rv = rinv_v[c, :]
                    acc0 = rows_v[r0 + 0, pl.ds(c0, _LANES)] * rv + mvec
                    acc1 = rows_v[r0 + 1, pl.ds(c0, _LANES)] * rv + mvec
                    for a in range(2, _ACTIVE, 2):
                        acc0 = acc0 + (rows_v[r0 + a, pl.ds(c0, _LANES)] * rv
                                       + mvec)
                        acc1 = acc1 + (rows_v[r0 + a + 1, pl.ds(c0, _LANES)]
                                       * rv + mvec)
                    total = (acc0 + acc1) - m20
                    out_v[p, pl.ds(c0, _LANES)] = (
                        total * sfin_v[c, :] + bias_v[c, :])

            pltpu.sync_copy(out_v, out_hbm.at[pl.ds(wid * _BPW + g * _G, _G)])

        start_gather(0, rows0, prow0, sem0, psem0)

        @pl.loop(0, _NGRP, step=2)
        def _(it):
            wait_gather(rows0, prow0, sem0, psem0)
            start_gather(it + 1, rows1, prow1, sem1, psem1)
            compute_group(it, rows0, prow0)

            wait_gather(rows1, prow1, sem1, psem1)

            @pl.when(it + 2 < _NGRP)
            def _():
                start_gather(it + 2, rows0, prow0, sem0, psem0)

            compute_group(it + 1, rows1, prow1)

    return body(idx_flat, weight, wp128, rinv33, sfin33, bias33)


def kernel(indices, weight, bias, scale_l1, scale_psqt):
    s_full = jnp.concatenate([scale_l1, scale_psqt]).astype(jnp.float32)
    rinv_full = (1.0 / s_full).astype(jnp.float32)
    starts = jnp.array([16 * c for c in range(32)] + [504], jnp.int32)
    cols = starts[:, None] + jnp.arange(16, dtype=jnp.int32)[None, :]
    rinv33 = jnp.take(rinv_full, cols, axis=0)
    sfin33 = jnp.take(s_full, cols, axis=0)
    bias33 = jnp.take(bias.astype(jnp.float32), cols, axis=0)
    idx_flat = indices.reshape(-1)
    wpsqt = weight.T[_N_L1:_TOTAL, :].T
    wp128 = jnp.pad(wpsqt, ((0, 0), (8, _PSQT_W - 8 - _N_PSQT)))
    return _sc_bag(idx_flat, weight, wp128, rinv33, sfin33, bias33)


# speedup vs baseline: 4.2337x; 1.5221x over previous
import dataclasses
import functools

import jax
import jax.numpy as jnp
from jax import lax
from jax.experimental import pallas as pl
from jax.experimental.pallas import tpu as pltpu
from jax.experimental.pallas import tpu_sc as plsc

_N_L1 = 512
_N_PSQT = 8
_TOTAL = _N_L1 + _N_PSQT
_ROWS = 100000
_BATCH = 4096
_ACTIVE = 20
_NC = 2
_NS = 16
_NW = _NC * _NS
_LANES = 16
_SB = 512
_NSB = _BATCH // _SB
_MAGIC = 12582912.0
_MAGIC20 = _MAGIC * _ACTIVE
_NCOL_HI = 17
_NCOL_LO = 16
_HI_WORKERS = _TOTAL - _NCOL_LO * _NW


def _sc_cols(idxT, wT, rinv, sfin, bias):
    mesh = plsc.VectorSubcoreMesh(core_axis_name="c", subcore_axis_name="s")
    cp = pltpu.CompilerParams()
    if "needs_layout_passes" in pltpu.CompilerParams.__dataclass_fields__:
        cp = dataclasses.replace(cp, needs_layout_passes=False)

    @functools.partial(
        pl.kernel,
        out_type=jax.ShapeDtypeStruct((_TOTAL, _BATCH), jnp.float32),
        mesh=mesh,
        compiler_params=cp,
        scratch_types=[
            pltpu.VMEM((_ROWS,), jnp.float32),
            pltpu.VMEM((_ACTIVE, _SB), jnp.int32),
            pltpu.VMEM((_ACTIVE, _SB), jnp.int32),
            pltpu.VMEM((_BATCH,), jnp.float32),
            pltpu.VMEM((_TOTAL,), jnp.float32),
            pltpu.VMEM((_TOTAL,), jnp.float32),
            pltpu.VMEM((_TOTAL,), jnp.float32),
            pltpu.SemaphoreType.DMA,
            pltpu.SemaphoreType.DMA,
            pltpu.SemaphoreType.DMA,
        ],
    )
    def body(idxT_hbm, wT_hbm, rinv_hbm, sfin_hbm, bias_hbm, outT_hbm,
             colbuf, idx0, idx1, outbuf, rinv_s, sfin_s, bias_s,
             csem, isem0, isem1):
        wid = lax.axis_index("s") * _NC + lax.axis_index("c")
        nt = jnp.where(wid < _HI_WORKERS, _NCOL_HI, _NCOL_LO)
        start = jnp.where(wid < _HI_WORKERS,
                          _NCOL_HI * wid,
                          _NCOL_LO * wid + _HI_WORKERS)
        pltpu.sync_copy(rinv_hbm, rinv_s)
        pltpu.sync_copy(sfin_hbm, sfin_s)
        pltpu.sync_copy(bias_hbm, bias_s)

        def idx_start(sb, buf, sem):
            pltpu.async_copy(idxT_hbm.at[:, pl.ds(sb * _SB, _SB)], buf, sem)

        def idx_wait(buf, sem):
            pltpu.make_async_copy(
                idxT_hbm.at[:, pl.ds(0, _SB)], buf, sem).wait()

        @pl.loop(0, _NCOL_HI)
        def _(k):
            @pl.when(k < nt)
            def _():
                c = start + k
                pltpu.async_copy(wT_hbm.at[c], colbuf, csem)
                idx_start(0, idx0, isem0)
                cvec = jnp.full((_LANES,), c, jnp.int32)
                rv = plsc.load_gather(rinv_s, [cvec])
                sv = plsc.load_gather(sfin_s, [cvec])
                bv = plsc.load_gather(bias_s, [cvec])
                mvec = jnp.full((_LANES,), _MAGIC, jnp.float32)
                m20v = jnp.full((_LANES,), _MAGIC20, jnp.float32)
                pltpu.make_async_copy(wT_hbm.at[c], colbuf, csem).wait()

                def do_block(sb, buf):
                    @pl.loop(0, _SB // _LANES)
                    def _(s16):
                        accs = [jnp.zeros((_LANES,), jnp.float32)
                                for _ in range(4)]
                        for a in range(_ACTIVE):
                            iv = buf[a, pl.ds(s16 * _LANES, _LANES)]
                            vals = plsc.load_gather(colbuf, [iv])
                            accs[a % 4] = accs[a % 4] + (vals * rv + mvec)
                        acc = (accs[0] + accs[1]) + (accs[2] + accs[3])
                        outbuf[pl.ds(sb * _SB + s16 * _LANES, _LANES)] = (
                            (acc - m20v) * sv + bv)

                for sb in range(_NSB):
                    cur, nxt = (idx0, idx1) if sb % 2 == 0 else (idx1, idx0)
                    csem_cur, csem_nxt = (
                        (isem0, isem1) if sb % 2 == 0 else (isem1, isem0))
                    idx_wait(cur, csem_cur)
                    if sb + 1 < _NSB:
                        idx_start(sb + 1, nxt, csem_nxt)
                    do_block(sb, cur)

                pltpu.sync_copy(outbuf, outT_hbm.at[c])

    return body(idxT, wT, rinv, sfin, bias)


def kernel(indices, weight, bias, scale_l1, scale_psqt):
    s_full = jnp.concatenate([scale_l1, scale_psqt]).astype(jnp.float32)
    rinv_full = (1.0 / s_full).astype(jnp.float32)
    wT = weight.T
    idxT = indices.T
    outT = _sc_cols(idxT, wT, rinv_full, s_full, bias.astype(jnp.float32))
    return outT.T
